# baseline (speedup 1.0000x reference)
import jax
import jax.numpy as jnp
from jax import lax
from jax.experimental import pallas as pl
from jax.experimental.pallas import tpu as pltpu

N = 10000
NA = 9984
NT = N - NA
BR = 80
NCHUNK = N // BR
NBUF = 8


def _mv_kernel(params_ref, xT_ref, adj_hbm, out_ref, buf_ref, tail_ref,
               sem_ref, tsem_ref):
    p = params_ref

    def copy_in(chunk, buf):
        pltpu.make_async_copy(
            adj_hbm.at[pl.ds(chunk * BR, BR), pl.ds(0, NA)],
            buf_ref.at[buf],
            sem_ref.at[buf],
        ).start()

    pltpu.make_async_copy(
        adj_hbm.at[:, pl.ds(NA, NT)], tail_ref, tsem_ref).start()
    for b in range(NBUF - 1):
        copy_in(b, b)

    vw = (p[0, 0] * xT_ref[0:1, 0:NA]
          + p[0, 1] * xT_ref[1:2, 0:NA]
          + p[0, 2] * xT_ref[2:3, 0:NA]
          + p[0, 3] * xT_ref[3:4, 0:NA]
          + p[0, 4])
    vt = (p[0, 0] * xT_ref[0:1, NA:N]
          + p[0, 1] * xT_ref[1:2, NA:N]
          + p[0, 2] * xT_ref[2:3, NA:N]
          + p[0, 3] * xT_ref[3:4, NA:N]
          + p[0, 4])

    pltpu.make_async_copy(
        adj_hbm.at[:, pl.ds(NA, NT)], tail_ref, tsem_ref).wait()

    def body(i, _):
        buf = lax.rem(i, NBUF)
        pltpu.make_async_copy(
            adj_hbm.at[pl.ds(i * BR, BR), pl.ds(0, NA)],
            buf_ref.at[buf], sem_ref.at[buf],
        ).wait()

        acc = jnp.sum(buf_ref[buf] * vw, axis=1, keepdims=True)
        acc = acc + jnp.sum(tail_ref[pl.ds(i * BR, BR), :] * vt,
                            axis=1, keepdims=True)
        out_ref[pl.ds(i * BR, BR), :] = acc + p[0, 5]

        @pl.when(i + NBUF - 1 < NCHUNK)
        def _():
            copy_in(i + NBUF - 1, lax.rem(i + NBUF - 1, NBUF))

        return 0

    lax.fori_loop(0, NCHUNK, body, 0)


@jax.jit
def kernel(x, adj, W_enc, b_enc, W_dec, b_dec):
    w = (W_dec @ W_enc).reshape(4)
    c = (b_enc @ W_dec.T).reshape(())
    params = jnp.concatenate(
        [w, c[None], b_dec.reshape(1)]).reshape(1, 6).astype(jnp.float32)
    xT = x.T

    out = pl.pallas_call(
        _mv_kernel,
        in_specs=[
            pl.BlockSpec(memory_space=pltpu.SMEM),
            pl.BlockSpec(memory_space=pltpu.VMEM),
            pl.BlockSpec(memory_space=pl.ANY),
        ],
        out_specs=pl.BlockSpec(memory_space=pltpu.VMEM),
        out_shape=jax.ShapeDtypeStruct((N, 1), jnp.float32),
        scratch_shapes=[
            pltpu.VMEM((NBUF, BR, NA), jnp.float32),
            pltpu.VMEM((N, NT), jnp.float32),
            pltpu.SemaphoreType.DMA((NBUF,)),
            pltpu.SemaphoreType.DMA,
        ],
    )(params, xT, adj)
    return out


# speedup vs baseline: 1.0019x; 1.0019x over previous
import jax
import jax.numpy as jnp
from jax import lax
from jax.experimental import pallas as pl
from jax.experimental.pallas import tpu as pltpu

N = 10000
NA = 9984
NT = N - NA
BR = 200
NCHUNK = N // BR
NBUF = 5


def _mv_kernel(params_ref, xT_ref, adj_hbm, out_ref, buf_ref, tail_ref,
               sem_ref, tsem_ref):
    p = params_ref

    def copy_in(chunk, buf):
        pltpu.make_async_copy(
            adj_hbm.at[pl.ds(chunk * BR, BR), pl.ds(0, NA)],
            buf_ref.at[buf],
            sem_ref.at[buf],
        ).start()

    pltpu.make_async_copy(
        adj_hbm.at[:, pl.ds(NA, NT)], tail_ref, tsem_ref).start()
    for b in range(NBUF - 1):
        copy_in(b, b)

    vw = (p[0, 0] * xT_ref[0:1, 0:NA]
          + p[0, 1] * xT_ref[1:2, 0:NA]
          + p[0, 2] * xT_ref[2:3, 0:NA]
          + p[0, 3] * xT_ref[3:4, 0:NA]
          + p[0, 4])
    vt = (p[0, 0] * xT_ref[0:1, NA:N]
          + p[0, 1] * xT_ref[1:2, NA:N]
          + p[0, 2] * xT_ref[2:3, NA:N]
          + p[0, 3] * xT_ref[3:4, NA:N]
          + p[0, 4])

    pltpu.make_async_copy(
        adj_hbm.at[:, pl.ds(NA, NT)], tail_ref, tsem_ref).wait()

    def body(i, _):
        buf = lax.rem(i, NBUF)
        pltpu.make_async_copy(
            adj_hbm.at[pl.ds(i * BR, BR), pl.ds(0, NA)],
            buf_ref.at[buf], sem_ref.at[buf],
        ).wait()

        acc = jnp.sum(buf_ref[buf] * vw, axis=1, keepdims=True)
        acc = acc + jnp.sum(tail_ref[pl.ds(i * BR, BR), :] * vt,
                            axis=1, keepdims=True)
        out_ref[pl.ds(i * BR, BR), :] = acc + p[0, 5]

        @pl.when(i + NBUF - 1 < NCHUNK)
        def _():
            copy_in(i + NBUF - 1, lax.rem(i + NBUF - 1, NBUF))

        return 0

    lax.fori_loop(0, NCHUNK, body, 0)


@jax.jit
def kernel(x, adj, W_enc, b_enc, W_dec, b_dec):
    w = (W_dec @ W_enc).reshape(4)
    c = (b_enc @ W_dec.T).reshape(())
    params = jnp.concatenate(
        [w, c[None], b_dec.reshape(1)]).reshape(1, 6).astype(jnp.float32)
    xT = x.T

    out = pl.pallas_call(
        _mv_kernel,
        in_specs=[
            pl.BlockSpec(memory_space=pltpu.SMEM),
            pl.BlockSpec(memory_space=pltpu.VMEM),
            pl.BlockSpec(memory_space=pl.ANY),
        ],
        out_specs=pl.BlockSpec(memory_space=pltpu.VMEM),
        out_shape=jax.ShapeDtypeStruct((N, 1), jnp.float32),
        scratch_shapes=[
            pltpu.VMEM((NBUF, BR, NA), jnp.float32),
            pltpu.VMEM((N, NT), jnp.float32),
            pltpu.SemaphoreType.DMA((NBUF,)),
            pltpu.SemaphoreType.DMA,
        ],
    )(params, xT, adj)
    return out
